# baseline (speedup 1.0000x reference)
import jax
import jax.numpy as jnp
from jax.experimental import pallas as pl
from jax.experimental.pallas import tpu as pltpu

_TM = 512
_TN = 512
_NJ = 4


def _mm_kernel(x_ref, w_ref, o_ref, wb_ref, xb_ref):
    i = pl.program_id(1)
    j = pl.program_id(2)

    @pl.when(i == 0)
    def _stage_w():
        wb_ref[:, pl.ds(j * _TN, _TN)] = w_ref[...].astype(jnp.bfloat16)

    @pl.when(j == 0)
    def _stage_x():
        xb_ref[...] = x_ref[...].astype(jnp.bfloat16)

    o_ref[...] = jnp.dot(
        xb_ref[...],
        wb_ref[:, pl.ds(j * _TN, _TN)],
        preferred_element_type=jnp.float32,
    )


def kernel(x, weight_t_padded):
    M, K = x.shape
    Kp, N = weight_t_padded.shape
    assert Kp == K and M % _TM == 0 and N % (_TN * _NJ) == 0, (M, K, Kp, N)

    grid = (N // (_TN * _NJ), M // _TM, _NJ)

    out = pl.pallas_call(
        _mm_kernel,
        out_shape=jax.ShapeDtypeStruct((M, N), jnp.float32),
        grid_spec=pltpu.PrefetchScalarGridSpec(
            num_scalar_prefetch=0,
            grid=grid,
            in_specs=[
                pl.BlockSpec((_TM, K), lambda n2, i, j: (i, 0)),
                pl.BlockSpec(
                    (K, _TN),
                    lambda n2, i, j: (
                        0,
                        jnp.where(i == 0, n2 * _NJ + j, n2 * _NJ + _NJ - 1),
                    ),
                ),
            ],
            out_specs=pl.BlockSpec(
                (_TM, _TN), lambda n2, i, j: (i, n2 * _NJ + j)
            ),
            scratch_shapes=[
                pltpu.VMEM((K, _TN * _NJ), jnp.bfloat16),
                pltpu.VMEM((_TM, K), jnp.bfloat16),
            ],
        ),
        compiler_params=pltpu.CompilerParams(
            dimension_semantics=("arbitrary", "arbitrary", "arbitrary"),
            vmem_limit_bytes=64 * 1024 * 1024,
        ),
        cost_estimate=pl.CostEstimate(
            flops=2 * M * K * N,
            transcendentals=0,
            bytes_accessed=2 * M * K * 4 + K * N * 4 + M * N * 4,
        ),
    )(x, weight_t_padded)
    return out


# speedup vs baseline: 1.0585x; 1.0585x over previous
import jax
import jax.numpy as jnp
from jax.experimental import pallas as pl
from jax.experimental.pallas import tpu as pltpu

_TM = 256
_NJ2 = 2


def _mm_kernel(x_ref, w_ref, o_ref):
    xb = x_ref[...].astype(jnp.bfloat16)
    o_ref[...] = jnp.dot(xb, w_ref[...], preferred_element_type=jnp.float32)


def kernel(x, weight_t_padded):
    M, K = x.shape
    Kp, N = weight_t_padded.shape
    assert Kp == K and M % _TM == 0 and N % _NJ2 == 0, (M, K, Kp, N)
    tn = N // _NJ2

    wb = weight_t_padded.astype(jnp.bfloat16)

    out = pl.pallas_call(
        _mm_kernel,
        out_shape=jax.ShapeDtypeStruct((M, N), jnp.float32),
        grid_spec=pltpu.PrefetchScalarGridSpec(
            num_scalar_prefetch=0,
            grid=(_NJ2, M // _TM),
            in_specs=[
                pl.BlockSpec((_TM, K), lambda j, i: (i, 0)),
                pl.BlockSpec((K, tn), lambda j, i: (0, j)),
            ],
            out_specs=pl.BlockSpec((_TM, tn), lambda j, i: (i, j)),
        ),
        compiler_params=pltpu.CompilerParams(
            dimension_semantics=("arbitrary", "arbitrary"),
            vmem_limit_bytes=64 * 1024 * 1024,
        ),
        cost_estimate=pl.CostEstimate(
            flops=2 * M * K * N,
            transcendentals=0,
            bytes_accessed=_NJ2 * M * K * 4 + K * N * 2 + M * N * 4,
        ),
    )(x, wb)
    return out


# speedup vs baseline: 1.1445x; 1.0812x over previous
import functools

import jax
import jax.numpy as jnp
from jax.experimental import pallas as pl
from jax.experimental.pallas import tpu as pltpu

_TM = 256


def _half_kernel_first(x_ref, w_ref, o_ref, wb_ref):
    @pl.when(pl.program_id(0) == 0)
    def _stage_w():
        wb_ref[...] = w_ref[...].astype(jnp.bfloat16)

    xb = x_ref[...].astype(jnp.bfloat16)
    o_ref[...] = jnp.dot(xb, wb_ref[...], preferred_element_type=jnp.float32)


def _half_kernel_second(x_ref, w_ref, prev_ref, o_ref, wb_ref):
    del prev_ref
    _half_kernel_first(x_ref, w_ref, o_ref, wb_ref)


def _half_call(x, w, prev, half):
    M, K = x.shape
    N = w.shape[1]
    tn = N // 2

    in_specs = [
        pl.BlockSpec((_TM, K), lambda i: (i, 0)),
        pl.BlockSpec((K, tn), functools.partial(lambda h, i: (0, h), half)),
    ]
    operands = (x, w)
    if prev is None:
        body, aliases = _half_kernel_first, {}
    else:
        in_specs.append(pl.BlockSpec(memory_space=pl.ANY))
        operands = (x, w, prev)
        body, aliases = _half_kernel_second, {2: 0}

    return pl.pallas_call(
        body,
        out_shape=jax.ShapeDtypeStruct((M, N), jnp.float32),
        grid_spec=pltpu.PrefetchScalarGridSpec(
            num_scalar_prefetch=0,
            grid=(M // _TM,),
            in_specs=in_specs,
            out_specs=pl.BlockSpec(
                (_TM, tn), functools.partial(lambda h, i: (i, h), half)
            ),
            scratch_shapes=[pltpu.VMEM((K, tn), jnp.bfloat16)],
        ),
        input_output_aliases=aliases,
        compiler_params=pltpu.CompilerParams(
            dimension_semantics=("arbitrary",),
            vmem_limit_bytes=64 * 1024 * 1024,
        ),
        cost_estimate=pl.CostEstimate(
            flops=2 * M * K * tn,
            transcendentals=0,
            bytes_accessed=M * K * 4 + K * tn * 4 + M * tn * 4,
        ),
    )(*operands)


def kernel(x, weight_t_padded):
    M, K = x.shape
    Kp, N = weight_t_padded.shape
    assert Kp == K and M % _TM == 0 and N % 256 == 0, (M, K, Kp, N)

    half0 = _half_call(x, weight_t_padded, None, 0)
    return _half_call(x, weight_t_padded, half0, 1)


# speedup vs baseline: 1.1752x; 1.0269x over previous
import jax
import jax.numpy as jnp
from jax.experimental import pallas as pl
from jax.experimental.pallas import tpu as pltpu

_TM = 256


def _left_kernel(x_ref, w_ref, wc_ref, o_ref, wbc_ref):
    wbc_ref[...] = wc_ref[...].astype(jnp.bfloat16)
    xb = x_ref[...].astype(jnp.bfloat16)
    wb = w_ref[...].astype(jnp.bfloat16)
    o_ref[...] = jnp.dot(xb, wb, preferred_element_type=jnp.float32)


def _right_kernel(x_ref, wb_ref, prev_ref, o_ref):
    del prev_ref
    xb = x_ref[...].astype(jnp.bfloat16)
    o_ref[...] = jnp.dot(xb, wb_ref[...], preferred_element_type=jnp.float32)


def kernel(x, weight_t_padded):
    M, K = x.shape
    Kp, N = weight_t_padded.shape
    tn = N // 2
    nsteps = M // _TM
    tc = tn // nsteps
    assert Kp == K and tc % 128 == 0, (M, K, Kp, N, tc)

    half0, wb1 = pl.pallas_call(
        _left_kernel,
        out_shape=[
            jax.ShapeDtypeStruct((M, N), jnp.float32),
            jax.ShapeDtypeStruct((K, tn), jnp.bfloat16),
        ],
        grid_spec=pltpu.PrefetchScalarGridSpec(
            num_scalar_prefetch=0,
            grid=(nsteps,),
            in_specs=[
                pl.BlockSpec((_TM, K), lambda i: (i, 0)),
                pl.BlockSpec((K, tn), lambda i: (0, 0)),
                pl.BlockSpec((K, tc), lambda i: (0, (tn // tc) + i)),
            ],
            out_specs=[
                pl.BlockSpec((_TM, tn), lambda i: (i, 0)),
                pl.BlockSpec((K, tc), lambda i: (0, i)),
            ],
        ),
        compiler_params=pltpu.CompilerParams(
            dimension_semantics=("arbitrary",),
            vmem_limit_bytes=64 * 1024 * 1024,
        ),
        cost_estimate=pl.CostEstimate(
            flops=2 * M * K * tn,
            transcendentals=0,
            bytes_accessed=M * K * 4 + K * N * 4 + M * tn * 4 + K * tn * 2,
        ),
    )(x, weight_t_padded, weight_t_padded)

    return pl.pallas_call(
        _right_kernel,
        out_shape=jax.ShapeDtypeStruct((M, N), jnp.float32),
        grid_spec=pltpu.PrefetchScalarGridSpec(
            num_scalar_prefetch=0,
            grid=(nsteps,),
            in_specs=[
                pl.BlockSpec((_TM, K), lambda i: (i, 0)),
                pl.BlockSpec((K, tn), lambda i: (0, 0)),
                pl.BlockSpec(memory_space=pl.ANY),
            ],
            out_specs=pl.BlockSpec((_TM, tn), lambda i: (i, 1)),
        ),
        input_output_aliases={2: 0},
        compiler_params=pltpu.CompilerParams(
            dimension_semantics=("arbitrary",),
            vmem_limit_bytes=64 * 1024 * 1024,
        ),
        cost_estimate=pl.CostEstimate(
            flops=2 * M * K * tn,
            transcendentals=0,
            bytes_accessed=M * K * 4 + K * tn * 2 + M * tn * 4,
        ),
    )(x, wb1, half0)
